# Initial kernel scaffold; baseline (speedup 1.0000x reference)
import jax
import jax.numpy as jnp
from jax import lax
from jax.experimental import pallas as pl
from jax.experimental.pallas import tpu as pltpu
from jax.experimental.pallas import tpu_sc as plsc

NN = 100000
NE = 6400000
NS = 100000
NC, NT = 2, 16
NW = NC * NT
EPW = NE // NW
B = 4000
NB = EPW // B
PAD = 100096
STRIPE = PAD // NT


def _sc_segsum(x, ptrs, csr):
    mesh = plsc.VectorSubcoreMesh(core_axis_name="c", subcore_axis_name="s",
                                  num_cores=NC, num_subcores=NT)

    def body(x_hbm, ptrs_hbm, csr_hbm, out_hbm, xloc, pb, cb, vb, acc):
        cid = lax.axis_index("c")
        sid = lax.axis_index("s")
        wid = sid * NC + cid

        pltpu.sync_copy(x_hbm, xloc)

        zeros = jnp.zeros((16,), jnp.float32)

        def zb(j, carry):
            vb[pl.ds(j * 16, 16)] = zeros
            return carry

        lax.fori_loop(0, B // 16, zb, 0)
        pltpu.sync_copy(vb, acc.at[pl.ds(sid * STRIPE, B)])
        pltpu.sync_copy(vb.at[pl.ds(0, STRIPE - B)],
                        acc.at[pl.ds(sid * STRIPE + B, STRIPE - B)])
        plsc.subcore_barrier()

        def blk(b, carry):
            base = wid * EPW + b * B
            pltpu.sync_copy(ptrs_hbm.at[pl.ds(base, B)], pb)
            pltpu.sync_copy(csr_hbm.at[pl.ds(base, B)], cb)

            def gath(j, c2):
                idx = pb[pl.ds(j * 16, 16)]
                vb[pl.ds(j * 16, 16)] = plsc.load_gather(xloc, [idx])
                return c2

            lax.fori_loop(0, B // 16, gath, 0)
            pltpu.sync_copy(vb, acc.at[cb], add=True)
            return carry

        lax.fori_loop(0, NB, blk, 0)
        plsc.subcore_barrier()

        pltpu.sync_copy(acc.at[pl.ds(sid * STRIPE, STRIPE)],
                        out_hbm.at[pl.ds(cid * PAD + sid * STRIPE, STRIPE)])

    return pl.kernel(
        body,
        out_type=jax.ShapeDtypeStruct((NC * PAD,), jnp.float32),
        mesh=mesh,
        scratch_types=[
            pltpu.VMEM((NN,), jnp.float32),
            pltpu.VMEM((B,), jnp.int32),
            pltpu.VMEM((B,), jnp.int32),
            pltpu.VMEM((B,), jnp.float32),
            pltpu.VMEM_SHARED((PAD,), jnp.float32),
        ],
    )(x, ptrs, csr)


def _tc_add(a_ref, b_ref, o_ref):
    o_ref[...] = a_ref[...] + b_ref[...]


def kernel(x, ptrs, csr):
    parts = _sc_segsum(x, ptrs, csr)
    a = parts[:PAD].reshape(PAD // 128, 128)
    b = parts[PAD:].reshape(PAD // 128, 128)
    out = pl.pallas_call(
        _tc_add,
        out_shape=jax.ShapeDtypeStruct((PAD // 128, 128), jnp.float32),
    )(a, b)
    return out.reshape(-1)[:NS]


# speedup vs baseline: 193.0234x; 193.0234x over previous
import jax
import jax.numpy as jnp
from jax import lax
from jax.experimental import pallas as pl
from jax.experimental.pallas import tpu as pltpu
from jax.experimental.pallas import tpu_sc as plsc

NN = 100000
NE = 6400000
NS = 100000
NC, NT = 2, 16
NW = NC * NT
EPW = NE // NW
B = 4000
NB = EPW // B
PAD = 100096
STRIPE = PAD // NT


def _sc_segsum(x, ptrs, csr):
    mesh = plsc.VectorSubcoreMesh(core_axis_name="c", subcore_axis_name="s",
                                  num_cores=NC, num_subcores=NT)

    def body(x_hbm, ptrs_hbm, csr_hbm, out_hbm, xloc, pb, cb, vb, acc):
        cid = lax.axis_index("c")
        sid = lax.axis_index("s")
        wid = sid * NC + cid

        pltpu.sync_copy(x_hbm, xloc)

        zeros = jnp.zeros((16,), jnp.float32)

        def zb(j, carry):
            vb[pl.ds(j * 16, 16)] = zeros
            return carry

        lax.fori_loop(0, B // 16, zb, 0)
        pltpu.sync_copy(vb, acc.at[pl.ds(sid * STRIPE, B)])
        pltpu.sync_copy(vb.at[pl.ds(0, STRIPE - B)],
                        acc.at[pl.ds(sid * STRIPE + B, STRIPE - B)])
        plsc.subcore_barrier()

        def blk(b, carry):
            base = wid * EPW + b * B
            pltpu.sync_copy(ptrs_hbm.at[pl.ds(base, B)], pb)
            pltpu.sync_copy(csr_hbm.at[pl.ds(base, B)], cb)

            def gath(j, c2):
                idx = pb[pl.ds(j * 16, 16)]
                vb[pl.ds(j * 16, 16)] = plsc.load_gather(xloc, [idx])
                return c2

            lax.fori_loop(0, B // 16, gath, 0)
            pltpu.sync_copy(vb, acc.at[cb], add=True)
            return carry

        lax.fori_loop(0, NB, blk, 0)
        plsc.subcore_barrier()

        pltpu.sync_copy(acc.at[pl.ds(sid * STRIPE, STRIPE)],
                        xloc.at[pl.ds(0, STRIPE)])
        pltpu.sync_copy(xloc.at[pl.ds(0, STRIPE)],
                        out_hbm.at[pl.ds(cid * PAD + sid * STRIPE, STRIPE)])

    return pl.kernel(
        body,
        out_type=jax.ShapeDtypeStruct((NC * PAD,), jnp.float32),
        mesh=mesh,
        compiler_params=pltpu.CompilerParams(needs_layout_passes=False),
        scratch_types=[
            pltpu.VMEM((NN,), jnp.float32),
            pltpu.VMEM((B,), jnp.int32),
            pltpu.VMEM((B,), jnp.int32),
            pltpu.VMEM((B,), jnp.float32),
            pltpu.VMEM_SHARED((PAD,), jnp.float32),
        ],
    )(x, ptrs, csr)


def _tc_add(a_ref, b_ref, o_ref):
    o_ref[...] = a_ref[...] + b_ref[...]


def kernel(x, ptrs, csr):
    parts = _sc_segsum(x, ptrs, csr)
    a = parts[:PAD].reshape(PAD // 128, 128)
    b = parts[PAD:].reshape(PAD // 128, 128)
    out = pl.pallas_call(
        _tc_add,
        out_shape=jax.ShapeDtypeStruct((PAD // 128, 128), jnp.float32),
    )(a, b)
    return out.reshape(-1)[:NS]


# speedup vs baseline: 426.6383x; 2.2103x over previous
import jax
import jax.numpy as jnp
from jax import lax
from jax.experimental import pallas as pl
from jax.experimental.pallas import tpu as pltpu
from jax.experimental.pallas import tpu_sc as plsc

NN = 100000
NE = 6400000
NS = 100000
NC, NT = 2, 16
NW = NC * NT
EPW = NE // NW
B = 2000
NB = EPW // B
NSLOT = 4
PAD = 100096
STRIPE = PAD // NT


def _sc_segsum(x, ptrs, csr):
    mesh = plsc.VectorSubcoreMesh(core_axis_name="c", subcore_axis_name="s",
                                  num_cores=NC, num_subcores=NT)

    def body(x_hbm, ptrs_hbm, csr_hbm, out_hbm, xloc,
             pb0, pb1, pb2, pb3, cb0, cb1, cb2, cb3, vb0, vb1, vb2, vb3,
             acc, sp0, sp1, sp2, sp3, sc0, sc1, sc2, sc3, sv0, sv1, sv2, sv3):
        cid = lax.axis_index("c")
        sid = lax.axis_index("s")
        wid = sid * NC + cid
        pbs, cbs, vbs = (pb0, pb1, pb2, pb3), (cb0, cb1, cb2, cb3), (vb0, vb1, vb2, vb3)
        sps, scs, svs = (sp0, sp1, sp2, sp3), (sc0, sc1, sc2, sc3), (sv0, sv1, sv2, sv3)

        def issue_in(slot, b):
            base = wid * EPW + b * B
            pltpu.async_copy(ptrs_hbm.at[pl.ds(base, B)], pbs[slot], sps[slot])
            pltpu.async_copy(csr_hbm.at[pl.ds(base, B)], cbs[slot], scs[slot])

        def wait_in(slot):
            pltpu.make_async_copy(ptrs_hbm.at[pl.ds(0, B)], pbs[slot], sps[slot]).wait()
            pltpu.make_async_copy(csr_hbm.at[pl.ds(0, B)], cbs[slot], scs[slot]).wait()

        def issue_sc(slot):
            pltpu.async_copy(vbs[slot], acc.at[cbs[slot]], svs[slot], add=True)

        def wait_sc(slot):
            pltpu.make_async_copy(vbs[slot], acc.at[cbs[slot]], svs[slot]).wait()

        issue_in(0, 0)
        pltpu.sync_copy(x_hbm, xloc)

        zeros = jnp.zeros((16,), jnp.float32)

        def zb(j, carry):
            vb3[pl.ds(j * 16, 16)] = zeros
            return carry

        lax.fori_loop(0, B // 16, zb, 0)
        pltpu.sync_copy(vb3, acc.at[pl.ds(sid * STRIPE, B)])
        pltpu.sync_copy(vb3, acc.at[pl.ds(sid * STRIPE + B, B)])
        pltpu.sync_copy(vb3, acc.at[pl.ds(sid * STRIPE + 2 * B, B)])
        pltpu.sync_copy(vb3.at[pl.ds(0, STRIPE - 3 * B)],
                        acc.at[pl.ds(sid * STRIPE + 3 * B, STRIPE - 3 * B)])
        plsc.subcore_barrier()

        def quad(i, carry):
            for phase in range(NSLOT):
                slot, nslot = phase, (phase + 1) % NSLOT
                b = NSLOT * i + phase
                wait_in(slot)

                @pl.when(b >= NSLOT - 1)
                def _():
                    wait_sc(nslot)

                @pl.when(b + 1 < NB)
                def _():
                    issue_in(nslot, b + 1)

                def gath(j, c2):
                    idx = pbs[slot][pl.ds(j * 16, 16)]
                    vbs[slot][pl.ds(j * 16, 16)] = plsc.load_gather(xloc, [idx])
                    return c2

                lax.fori_loop(0, B // 16, gath, 0, unroll=4)
                issue_sc(slot)
            return carry

        lax.fori_loop(0, NB // NSLOT, quad, 0)
        for q in range(NB - NSLOT + 1, NB):
            wait_sc(q % NSLOT)
        plsc.subcore_barrier()

        pltpu.sync_copy(acc.at[pl.ds(sid * STRIPE, STRIPE)],
                        xloc.at[pl.ds(0, STRIPE)])
        pltpu.sync_copy(xloc.at[pl.ds(0, STRIPE)],
                        out_hbm.at[pl.ds(cid * PAD + sid * STRIPE, STRIPE)])

    return pl.kernel(
        body,
        out_type=jax.ShapeDtypeStruct((NC * PAD,), jnp.float32),
        mesh=mesh,
        compiler_params=pltpu.CompilerParams(needs_layout_passes=False),
        scratch_types=(
            [pltpu.VMEM((NN,), jnp.float32)]
            + [pltpu.VMEM((B,), jnp.int32) for _ in range(NSLOT)]
            + [pltpu.VMEM((B,), jnp.int32) for _ in range(NSLOT)]
            + [pltpu.VMEM((B,), jnp.float32) for _ in range(NSLOT)]
            + [pltpu.VMEM_SHARED((PAD,), jnp.float32)]
            + [pltpu.SemaphoreType.DMA for _ in range(3 * NSLOT)]
        ),
    )(x, ptrs, csr)


def _tc_add(a_ref, b_ref, o_ref):
    o_ref[...] = a_ref[...] + b_ref[...]


def kernel(x, ptrs, csr):
    parts = _sc_segsum(x, ptrs, csr)
    a = parts[:PAD].reshape(PAD // 128, 128)
    b = parts[PAD:].reshape(PAD // 128, 128)
    out = pl.pallas_call(
        _tc_add,
        out_shape=jax.ShapeDtypeStruct((PAD // 128, 128), jnp.float32),
    )(a, b)
    return out.reshape(-1)[:NS]
